# baseline (speedup 1.0000x reference)
import functools

import jax
import jax.numpy as jnp
from jax import lax
from jax.experimental import pallas as pl
from jax.experimental.pallas import tpu as pltpu
from jax.experimental.pallas import tpu_sc as plsc

_LANES = 16
_HIMASK = -65536


def _unpk(w):
    lo = plsc.bitcast(jnp.left_shift(w, 16), jnp.float32)
    hi = plsc.bitcast(jnp.bitwise_and(w, jnp.int32(_HIMASK)), jnp.float32)
    return lo, hi


def _pk(lo, hi):
    lw = lax.shift_right_logical(plsc.bitcast(lo, jnp.int32), 16)
    hw = jnp.bitwise_and(plsc.bitcast(hi, jnp.int32), jnp.int32(_HIMASK))
    return jnp.bitwise_or(lw, hw)


def _anyv(x):
    return plsc.all_reduce_population_count(x)[0] > 0


def _tc_node_body(xT_ref, posT_ref, Wx_ref, Wp_ref, b_ref, uP_ref, vT_ref, p3T_ref):
    pz = posT_ref[0:1, :]
    phi = posT_ref[1:2, :]
    px = jnp.cos(phi)
    py = jnp.sin(phi)
    p3T_ref[...] = jnp.concatenate([px, py, pz], axis=0)
    Wp = Wp_ref[...]
    v = Wp[:, 0:1] * px + Wp[:, 1:2] * py + Wp[:, 2:3] * pz
    vT_ref[...] = v
    u = (
        jnp.dot(Wx_ref[...], xT_ref[...], preferred_element_type=jnp.float32)
        + v
        + b_ref[...]
    )
    D = u.shape[0]
    H = D // 2
    lo = lax.bitcast_convert_type(u[:H].astype(jnp.bfloat16), jnp.uint16)
    hi = lax.bitcast_convert_type(u[H:].astype(jnp.bfloat16), jnp.uint16)
    packed = lo.astype(jnp.uint32) | (hi.astype(jnp.uint32) << 16)
    uP_ref[...] = lax.bitcast_convert_type(packed, jnp.int32)


def _node_transform(xT, posT, Wx, Wp, b2, NP, D, BN):
    grid = (NP // BN,)
    return pl.pallas_call(
        _tc_node_body,
        grid=grid,
        in_specs=[
            pl.BlockSpec((D, BN), lambda j: (0, j)),
            pl.BlockSpec((2, BN), lambda j: (0, j)),
            pl.BlockSpec((D, D), lambda j: (0, 0)),
            pl.BlockSpec((D, 3), lambda j: (0, 0)),
            pl.BlockSpec((D, 1), lambda j: (0, 0)),
        ],
        out_specs=[
            pl.BlockSpec((D // 2, BN), lambda j: (0, j)),
            pl.BlockSpec((D, BN), lambda j: (0, j)),
            pl.BlockSpec((3, BN), lambda j: (0, j)),
        ],
        out_shape=[
            jax.ShapeDtypeStruct((D // 2, NP), jnp.int32),
            jax.ShapeDtypeStruct((D, NP), jnp.float32),
            jax.ShapeDtypeStruct((3, NP), jnp.float32),
        ],
    )(xT, posT, Wx, Wp, b2)


def _make_sc_edge(D, NP, E, CH, NC, NS):
    H = D // 2
    PPT = H // NS
    ES = E // NC
    U = 2
    mesh = plsc.VectorSubcoreMesh(
        core_axis_name="c", subcore_axis_name="s", num_cores=NC, num_subcores=NS
    )

    @functools.partial(
        pl.kernel,
        out_type=jax.ShapeDtypeStruct((NC * H * NP,), jnp.int32),
        mesh=mesh,
        compiler_params=pltpu.CompilerParams(needs_layout_passes=False),
        scratch_types=[
            pltpu.VMEM((PPT * NP,), jnp.int32),
            pltpu.VMEM((PPT * NP,), jnp.int32),
            pltpu.VMEM((NP,), jnp.int32),
            pltpu.VMEM((CH,), jnp.int32),
            pltpu.VMEM((CH,), jnp.int32),
            pltpu.VMEM((CH,), jnp.int32),
            pltpu.VMEM((CH,), jnp.int32),
            pltpu.SemaphoreType.DMA,
            pltpu.SemaphoreType.DMA,
            pltpu.SemaphoreType.DMA,
            pltpu.SemaphoreType.DMA,
        ],
    )
    def agg(
        uP_hbm,
        src_hbm,
        dst_hbm,
        part_hbm,
        u_v,
        acc_v,
        tag_v,
        src_a,
        src_b,
        dst_a,
        dst_b,
        sem_sa,
        sem_sb,
        sem_da,
        sem_db,
    ):
        shard = lax.axis_index("c")
        grp = lax.axis_index("s")
        fbase = grp * (PPT * NP)
        pltpu.sync_copy(uP_hbm.at[pl.ds(fbase, PPT * NP)], u_v)
        pltpu.sync_copy(uP_hbm.at[pl.ds(fbase, PPT * NP)], acc_v)
        ebase = shard * ES

        def process(src_v, dst_v, cbase):
            def vec_body(i, _):
                iota16 = lax.broadcasted_iota(jnp.int32, (_LANES,), 0)
                gbase = cbase + i * (U * _LANES)
                d16s, codes = [], []
                idx_d, gw, aw = [], [], []
                for k in range(U):
                    base = (i * U + k) * _LANES
                    s16 = src_v[pl.ds(base, _LANES)]
                    d16 = dst_v[pl.ds(base, _LANES)]
                    code = gbase + jnp.int32(k * _LANES) + iota16
                    plsc.store_scatter(tag_v, [d16], code)
                    d16s.append(d16)
                    codes.append(code)
                    for p in range(PPT):
                        idx_d.append(d16 + jnp.int32(p * NP))
                        gw.append(plsc.load_gather(u_v, [s16 + jnp.int32(p * NP)]))
                for j in range(U * PPT):
                    aw.append(plsc.load_gather(acc_v, [idx_d[j]]))
                mw = []
                for j in range(U * PPT):
                    g_lo, g_hi = _unpk(gw[j])
                    a_lo, a_hi = _unpk(aw[j])
                    mw.append(
                        _pk(jnp.maximum(g_lo, a_lo), jnp.maximum(g_hi, a_hi))
                    )
                for j in range(U * PPT):
                    plsc.store_scatter(acc_v, [idx_d[j]], mw[j])
                tags = [plsc.load_gather(tag_v, [d16s[k]]) for k in range(U)]
                dup = tags[0] != codes[0]
                for k in range(1, U):
                    dup = dup | (tags[k] != codes[k])

                @pl.when(_anyv(dup))
                def _repair():
                    bad = None
                    for j in range(U * PPT):
                        rl, rh = _unpk(plsc.load_gather(acc_v, [idx_d[j]]))
                        ml, mh = _unpk(mw[j])
                        c = (ml > rl) | (mh > rh)
                        bad = c if bad is None else (bad | c)

                    def fix_cond(dirty):
                        return dirty

                    def fix_body(_):
                        dirty = jnp.bool_(False)
                        for j in range(U * PPT):
                            rl, rh = _unpk(plsc.load_gather(acc_v, [idx_d[j]]))
                            ml, mh = _unpk(mw[j])
                            need = (ml > rl) | (mh > rh)
                            wl = jnp.maximum(ml, rl)
                            wh = jnp.maximum(mh, rh)
                            plsc.store_scatter(
                                acc_v, [idx_d[j]], _pk(wl, wh), mask=need
                            )
                            dirty = dirty | _anyv(need)
                        return dirty

                    lax.while_loop(fix_cond, fix_body, _anyv(bad))

                return 0

            lax.fori_loop(0, CH // (U * _LANES), vec_body, 0)

        nchunks = ES // CH
        bufs = [(src_a, dst_a, sem_sa, sem_da), (src_b, dst_b, sem_sb, sem_db)]
        for bi, (sb, db, ss, sd) in enumerate(bufs):
            pltpu.async_copy(src_hbm.at[pl.ds(ebase + bi * CH, CH)], sb, ss)
            pltpu.async_copy(dst_hbm.at[pl.ds(ebase + bi * CH, CH)], db, sd)

        def outer_body(oi, _):
            for bi, (sb, db, ss, sd) in enumerate(bufs):
                ci = oi * 2 + bi
                pltpu.make_async_copy(
                    src_hbm.at[pl.ds(ebase + ci * CH, CH)], sb, ss
                ).wait()
                pltpu.make_async_copy(
                    dst_hbm.at[pl.ds(ebase + ci * CH, CH)], db, sd
                ).wait()
                process(sb, db, ci * CH)
                nci = lax.rem(ci + 2, nchunks)
                pltpu.async_copy(src_hbm.at[pl.ds(ebase + nci * CH, CH)], sb, ss)
                pltpu.async_copy(dst_hbm.at[pl.ds(ebase + nci * CH, CH)], db, sd)
            return 0

        lax.fori_loop(0, nchunks // 2, outer_body, 0)
        for bi, (sb, db, ss, sd) in enumerate(bufs):
            pltpu.make_async_copy(src_hbm.at[pl.ds(ebase + bi * CH, CH)], sb, ss).wait()
            pltpu.make_async_copy(dst_hbm.at[pl.ds(ebase + bi * CH, CH)], db, sd).wait()

        pltpu.sync_copy(acc_v, part_hbm.at[pl.ds(shard * (H * NP) + fbase, PPT * NP)])

    return agg


def _make_sc_merge(D, NP, NC, NS):
    H = D // 2
    NW = NC * NS
    PR = H // NW
    mesh = plsc.VectorSubcoreMesh(
        core_axis_name="c", subcore_axis_name="s", num_cores=NC, num_subcores=NS
    )

    @functools.partial(
        pl.kernel,
        out_type=jax.ShapeDtypeStruct((D * NP,), jnp.float32),
        mesh=mesh,
        compiler_params=pltpu.CompilerParams(needs_layout_passes=False),
        scratch_types=[
            pltpu.VMEM((PR * NP,), jnp.int32),
            pltpu.VMEM((PR * NP,), jnp.int32),
            pltpu.VMEM((2 * PR * NP,), jnp.float32),
        ],
    )
    def merge(part_hbm, vT_hbm, out_hbm, a0_v, a1_v, v_v):
        cid = lax.axis_index("c")
        sid = lax.axis_index("s")
        wid = sid * NC + cid
        pbase = wid * (PR * NP)
        pltpu.sync_copy(part_hbm.at[pl.ds(pbase, PR * NP)], a0_v)
        pltpu.sync_copy(part_hbm.at[pl.ds(H * NP + pbase, PR * NP)], a1_v)
        pltpu.sync_copy(vT_hbm.at[pl.ds(pbase, PR * NP)], v_v.at[pl.ds(0, PR * NP)])
        pltpu.sync_copy(
            vT_hbm.at[pl.ds(H * NP + pbase, PR * NP)],
            v_v.at[pl.ds(PR * NP, PR * NP)],
        )

        def body(i, _):
            sl = pl.ds(i * _LANES, _LANES)
            slh = pl.ds(PR * NP + i * _LANES, _LANES)
            l0, h0 = _unpk(a0_v[sl])
            l1, h1 = _unpk(a1_v[sl])
            v_v[sl] = jnp.maximum(l0, l1) - v_v[sl]
            v_v[slh] = jnp.maximum(h0, h1) - v_v[slh]
            return 0

        lax.fori_loop(0, PR * NP // _LANES, body, 0)
        pltpu.sync_copy(v_v.at[pl.ds(0, PR * NP)], out_hbm.at[pl.ds(pbase, PR * NP)])
        pltpu.sync_copy(
            v_v.at[pl.ds(PR * NP, PR * NP)],
            out_hbm.at[pl.ds(H * NP + pbase, PR * NP)],
        )

    return merge


def kernel(x, pos, edge_index, batch, W, b):
    N, D = x.shape
    E = edge_index.shape[1]
    NC, NS = 2, 16
    NW = NC * NS
    assert (D // 2) % NW == 0
    NP = -(-N // 256) * 256

    ES = E // NC
    CH = 0
    for cand in range(4096, 31, -32):
        if ES % cand == 0 and (ES // cand) % 2 == 0:
            CH = cand
            break
    assert CH > 0

    xT = jnp.pad(x.T, ((0, 0), (0, NP - N)))
    posT = jnp.pad(pos.T, ((0, 0), (0, NP - N)))
    Wx = W[:, :D]
    Wp = W[:, D:]
    b2 = b[:, None]

    uP, vT, p3T = _node_transform(xT, posT, Wx, Wp, b2, NP, D, 512)

    src = edge_index[0]
    dst = edge_index[1]
    agg = _make_sc_edge(D, NP, E, CH, NC, NS)
    part = agg(uP.reshape((D // 2) * NP), src, dst)
    merge = _make_sc_merge(D, NP, NC, NS)
    outF = merge(part, vT.reshape(D * NP))

    out = outF.reshape(D, NP)[:, :N].T
    pos3 = p3T[:, :N].T
    return (out, pos3, batch)


# speedup vs baseline: 1.2373x; 1.2373x over previous
import functools

import jax
import jax.numpy as jnp
from jax import lax
from jax.experimental import pallas as pl
from jax.experimental.pallas import tpu as pltpu
from jax.experimental.pallas import tpu_sc as plsc

_LANES = 16
_HIMASK = -65536


def _unpk(w):
    lo = plsc.bitcast(jnp.left_shift(w, 16), jnp.float32)
    hi = plsc.bitcast(jnp.bitwise_and(w, jnp.int32(_HIMASK)), jnp.float32)
    return lo, hi


def _pk(lo, hi):
    lw = lax.shift_right_logical(plsc.bitcast(lo, jnp.int32), 16)
    hw = jnp.bitwise_and(plsc.bitcast(hi, jnp.int32), jnp.int32(_HIMASK))
    return jnp.bitwise_or(lw, hw)


def _anyv(x):
    return plsc.all_reduce_population_count(x)[0] > 0


def _tc_node_body(x_ref, posT_ref, Wx_ref, Wp_ref, b_ref, uP_ref, vT_ref, p3T_ref):
    pz = posT_ref[0:1, :]
    phi = posT_ref[1:2, :]
    px = jnp.cos(phi)
    py = jnp.sin(phi)
    p3T_ref[...] = jnp.concatenate([px, py, pz], axis=0)
    Wp = Wp_ref[...]
    v = Wp[:, 0:1] * px + Wp[:, 1:2] * py + Wp[:, 2:3] * pz
    vT_ref[...] = v
    u = (
        lax.dot_general(
            Wx_ref[...],
            x_ref[...],
            (((1,), (1,)), ((), ())),
            preferred_element_type=jnp.float32,
        )
        + v
        + b_ref[...]
    )
    D = u.shape[0]
    H = D // 2
    lo = lax.bitcast_convert_type(u[:H].astype(jnp.bfloat16), jnp.uint16)
    hi = lax.bitcast_convert_type(u[H:].astype(jnp.bfloat16), jnp.uint16)
    packed = lo.astype(jnp.uint32) | (hi.astype(jnp.uint32) << 16)
    uP_ref[...] = lax.bitcast_convert_type(packed, jnp.int32)


def _node_transform(xp, posT, Wx, Wp, b2, NP, D, BN):
    grid = (NP // BN,)
    return pl.pallas_call(
        _tc_node_body,
        grid=grid,
        in_specs=[
            pl.BlockSpec((BN, D), lambda j: (j, 0)),
            pl.BlockSpec((2, BN), lambda j: (0, j)),
            pl.BlockSpec((D, D), lambda j: (0, 0)),
            pl.BlockSpec((D, 3), lambda j: (0, 0)),
            pl.BlockSpec((D, 1), lambda j: (0, 0)),
        ],
        out_specs=[
            pl.BlockSpec((D // 2, BN), lambda j: (0, j)),
            pl.BlockSpec((D, BN), lambda j: (0, j)),
            pl.BlockSpec((3, BN), lambda j: (0, j)),
        ],
        out_shape=[
            jax.ShapeDtypeStruct((D // 2, NP), jnp.int32),
            jax.ShapeDtypeStruct((D, NP), jnp.float32),
            jax.ShapeDtypeStruct((3, NP), jnp.float32),
        ],
    )(xp, posT, Wx, Wp, b2)


def _make_sc_edge(D, NP, E, CH, NC, NS):
    H = D // 2
    PPT = H // NS
    ES = E // NC
    U = 2
    mesh = plsc.VectorSubcoreMesh(
        core_axis_name="c", subcore_axis_name="s", num_cores=NC, num_subcores=NS
    )

    @functools.partial(
        pl.kernel,
        out_type=jax.ShapeDtypeStruct((NC * H * NP,), jnp.int32),
        mesh=mesh,
        compiler_params=pltpu.CompilerParams(needs_layout_passes=False),
        scratch_types=[
            pltpu.VMEM((PPT * NP,), jnp.int32),
            pltpu.VMEM((PPT * NP,), jnp.int32),
            pltpu.VMEM((CH,), jnp.int32),
            pltpu.VMEM((CH,), jnp.int32),
            pltpu.VMEM((CH,), jnp.int32),
            pltpu.VMEM((CH,), jnp.int32),
            pltpu.SemaphoreType.DMA,
            pltpu.SemaphoreType.DMA,
            pltpu.SemaphoreType.DMA,
            pltpu.SemaphoreType.DMA,
        ],
    )
    def agg(
        uP_hbm,
        src_hbm,
        dst_hbm,
        part_hbm,
        u_v,
        acc_v,
        src_a,
        src_b,
        dst_a,
        dst_b,
        sem_sa,
        sem_sb,
        sem_da,
        sem_db,
    ):
        shard = lax.axis_index("c")
        grp = lax.axis_index("s")
        fbase = grp * (PPT * NP)
        pltpu.sync_copy(uP_hbm.at[pl.ds(fbase, PPT * NP)], u_v)
        pltpu.sync_copy(uP_hbm.at[pl.ds(fbase, PPT * NP)], acc_v)
        ebase = shard * ES

        def process(src_v, dst_v):
            def vec_body(i, _):
                idx_d, gw, aw = [], [], []
                for k in range(U):
                    base = (i * U + k) * _LANES
                    s16 = src_v[pl.ds(base, _LANES)]
                    d16 = dst_v[pl.ds(base, _LANES)]
                    for p in range(PPT):
                        idx_d.append(d16 + jnp.int32(p * NP))
                        gw.append(plsc.load_gather(u_v, [s16 + jnp.int32(p * NP)]))
                for j in range(U * PPT):
                    aw.append(plsc.load_gather(acc_v, [idx_d[j]]))
                mw = []
                for j in range(U * PPT):
                    g_lo, g_hi = _unpk(gw[j])
                    a_lo, a_hi = _unpk(aw[j])
                    mw.append(
                        _pk(jnp.maximum(g_lo, a_lo), jnp.maximum(g_hi, a_hi))
                    )
                for j in range(U * PPT):
                    plsc.store_scatter(acc_v, [idx_d[j]], mw[j])
                bad = None
                for j in range(U * PPT):
                    rl, rh = _unpk(plsc.load_gather(acc_v, [idx_d[j]]))
                    ml, mh = _unpk(mw[j])
                    c = (ml > rl) | (mh > rh)
                    bad = c if bad is None else (bad | c)

                def fix_cond(dirty):
                    return dirty

                def fix_body(_):
                    dirty = jnp.bool_(False)
                    for j in range(U * PPT):
                        rl, rh = _unpk(plsc.load_gather(acc_v, [idx_d[j]]))
                        ml, mh = _unpk(mw[j])
                        need = (ml > rl) | (mh > rh)
                        wl = jnp.maximum(ml, rl)
                        wh = jnp.maximum(mh, rh)
                        plsc.store_scatter(
                            acc_v, [idx_d[j]], _pk(wl, wh), mask=need
                        )
                        dirty = dirty | _anyv(need)
                    return dirty

                lax.while_loop(fix_cond, fix_body, _anyv(bad))
                return 0

            lax.fori_loop(0, CH // (U * _LANES), vec_body, 0)

        nchunks = ES // CH
        bufs = [(src_a, dst_a, sem_sa, sem_da), (src_b, dst_b, sem_sb, sem_db)]
        for bi, (sb, db, ss, sd) in enumerate(bufs):
            pltpu.async_copy(src_hbm.at[pl.ds(ebase + bi * CH, CH)], sb, ss)
            pltpu.async_copy(dst_hbm.at[pl.ds(ebase + bi * CH, CH)], db, sd)

        def outer_body(oi, _):
            for bi, (sb, db, ss, sd) in enumerate(bufs):
                ci = oi * 2 + bi
                pltpu.make_async_copy(
                    src_hbm.at[pl.ds(ebase + ci * CH, CH)], sb, ss
                ).wait()
                pltpu.make_async_copy(
                    dst_hbm.at[pl.ds(ebase + ci * CH, CH)], db, sd
                ).wait()
                process(sb, db)
                nci = lax.rem(ci + 2, nchunks)
                pltpu.async_copy(src_hbm.at[pl.ds(ebase + nci * CH, CH)], sb, ss)
                pltpu.async_copy(dst_hbm.at[pl.ds(ebase + nci * CH, CH)], db, sd)
            return 0

        lax.fori_loop(0, nchunks // 2, outer_body, 0)
        for bi, (sb, db, ss, sd) in enumerate(bufs):
            pltpu.make_async_copy(src_hbm.at[pl.ds(ebase + bi * CH, CH)], sb, ss).wait()
            pltpu.make_async_copy(dst_hbm.at[pl.ds(ebase + bi * CH, CH)], db, sd).wait()

        pltpu.sync_copy(acc_v, part_hbm.at[pl.ds(shard * (H * NP) + fbase, PPT * NP)])

    return agg


def _make_sc_merge(D, NP, NC, NS):
    H = D // 2
    NW = NC * NS
    PR = H // NW
    mesh = plsc.VectorSubcoreMesh(
        core_axis_name="c", subcore_axis_name="s", num_cores=NC, num_subcores=NS
    )

    @functools.partial(
        pl.kernel,
        out_type=jax.ShapeDtypeStruct((D * NP,), jnp.float32),
        mesh=mesh,
        compiler_params=pltpu.CompilerParams(needs_layout_passes=False),
        scratch_types=[
            pltpu.VMEM((PR * NP,), jnp.int32),
            pltpu.VMEM((PR * NP,), jnp.int32),
            pltpu.VMEM((2 * PR * NP,), jnp.float32),
        ],
    )
    def merge(part_hbm, vT_hbm, out_hbm, a0_v, a1_v, v_v):
        cid = lax.axis_index("c")
        sid = lax.axis_index("s")
        wid = sid * NC + cid
        pbase = wid * (PR * NP)
        pltpu.sync_copy(part_hbm.at[pl.ds(pbase, PR * NP)], a0_v)
        pltpu.sync_copy(part_hbm.at[pl.ds(H * NP + pbase, PR * NP)], a1_v)
        pltpu.sync_copy(vT_hbm.at[pl.ds(pbase, PR * NP)], v_v.at[pl.ds(0, PR * NP)])
        pltpu.sync_copy(
            vT_hbm.at[pl.ds(H * NP + pbase, PR * NP)],
            v_v.at[pl.ds(PR * NP, PR * NP)],
        )

        def body(i, _):
            sl = pl.ds(i * _LANES, _LANES)
            slh = pl.ds(PR * NP + i * _LANES, _LANES)
            l0, h0 = _unpk(a0_v[sl])
            l1, h1 = _unpk(a1_v[sl])
            v_v[sl] = jnp.maximum(l0, l1) - v_v[sl]
            v_v[slh] = jnp.maximum(h0, h1) - v_v[slh]
            return 0

        lax.fori_loop(0, PR * NP // _LANES, body, 0)
        pltpu.sync_copy(v_v.at[pl.ds(0, PR * NP)], out_hbm.at[pl.ds(pbase, PR * NP)])
        pltpu.sync_copy(
            v_v.at[pl.ds(PR * NP, PR * NP)],
            out_hbm.at[pl.ds(H * NP + pbase, PR * NP)],
        )

    return merge


def kernel(x, pos, edge_index, batch, W, b):
    N, D = x.shape
    E = edge_index.shape[1]
    NC, NS = 2, 16
    NW = NC * NS
    assert (D // 2) % NW == 0
    NP = -(-N // 256) * 256

    ES = E // NC
    CH = 0
    for cand in range(8192, 31, -32):
        if ES % cand == 0 and (ES // cand) % 2 == 0:
            CH = cand
            break
    assert CH > 0

    xp = jnp.pad(x, ((0, NP - N), (0, 0)))
    posT = jnp.pad(pos.T, ((0, 0), (0, NP - N)))
    Wx = W[:, :D]
    Wp = W[:, D:]
    b2 = b[:, None]

    uP, vT, p3T = _node_transform(xp, posT, Wx, Wp, b2, NP, D, 512)

    src = edge_index[0]
    dst = edge_index[1]
    agg = _make_sc_edge(D, NP, E, CH, NC, NS)
    part = agg(uP.reshape((D // 2) * NP), src, dst)
    merge = _make_sc_merge(D, NP, NC, NS)
    outF = merge(part, vT.reshape(D * NP))

    out = outF.reshape(D, NP)[:, :N].T
    pos3 = p3T[:, :N].T
    return (out, pos3, batch)


# speedup vs baseline: 1.3049x; 1.0547x over previous
import functools

import jax
import jax.numpy as jnp
from jax import lax
from jax.experimental import pallas as pl
from jax.experimental.pallas import tpu as pltpu
from jax.experimental.pallas import tpu_sc as plsc

_LANES = 16
_HIMASK = -65536


def _unpk(w):
    lo = plsc.bitcast(jnp.left_shift(w, 16), jnp.float32)
    hi = plsc.bitcast(jnp.bitwise_and(w, jnp.int32(_HIMASK)), jnp.float32)
    return lo, hi


def _pk(lo, hi):
    lw = lax.shift_right_logical(plsc.bitcast(lo, jnp.int32), 16)
    hw = jnp.bitwise_and(plsc.bitcast(hi, jnp.int32), jnp.int32(_HIMASK))
    return jnp.bitwise_or(lw, hw)


def _anyv(x):
    return plsc.all_reduce_population_count(x)[0] > 0


def _tf16(w):
    w16 = plsc.bitcast(w, jnp.int16)
    key = jnp.where(w16 < jnp.int16(0), w16 ^ jnp.int16(0x7FFF), w16)
    return plsc.bitcast(key, jnp.int32)


def _maxw(a, b):
    return plsc.bitcast(
        jnp.maximum(plsc.bitcast(a, jnp.int16), plsc.bitcast(b, jnp.int16)),
        jnp.int32,
    )


def _tc_node_body(x_ref, posT_ref, Wx_ref, Wp_ref, b_ref, uP_ref, vT_ref, p3T_ref):
    pz = posT_ref[0:1, :]
    phi = posT_ref[1:2, :]
    px = jnp.cos(phi)
    py = jnp.sin(phi)
    p3T_ref[...] = jnp.concatenate([px, py, pz], axis=0)
    Wp = Wp_ref[...]
    v = Wp[:, 0:1] * px + Wp[:, 1:2] * py + Wp[:, 2:3] * pz
    vT_ref[...] = v
    u = (
        lax.dot_general(
            Wx_ref[...],
            x_ref[...],
            (((1,), (1,)), ((), ())),
            preferred_element_type=jnp.float32,
        )
        + v
        + b_ref[...]
    )
    D = u.shape[0]
    H = D // 2
    lo = lax.bitcast_convert_type(u[:H].astype(jnp.bfloat16), jnp.uint16)
    hi = lax.bitcast_convert_type(u[H:].astype(jnp.bfloat16), jnp.uint16)
    packed = lo.astype(jnp.uint32) | (hi.astype(jnp.uint32) << 16)
    uP_ref[...] = lax.bitcast_convert_type(packed, jnp.int32)


def _node_transform(xp, posT, Wx, Wp, b2, NP, D, BN):
    grid = (NP // BN,)
    return pl.pallas_call(
        _tc_node_body,
        grid=grid,
        in_specs=[
            pl.BlockSpec((BN, D), lambda j: (j, 0)),
            pl.BlockSpec((2, BN), lambda j: (0, j)),
            pl.BlockSpec((D, D), lambda j: (0, 0)),
            pl.BlockSpec((D, 3), lambda j: (0, 0)),
            pl.BlockSpec((D, 1), lambda j: (0, 0)),
        ],
        out_specs=[
            pl.BlockSpec((D // 2, BN), lambda j: (0, j)),
            pl.BlockSpec((D, BN), lambda j: (0, j)),
            pl.BlockSpec((3, BN), lambda j: (0, j)),
        ],
        out_shape=[
            jax.ShapeDtypeStruct((D // 2, NP), jnp.int32),
            jax.ShapeDtypeStruct((D, NP), jnp.float32),
            jax.ShapeDtypeStruct((3, NP), jnp.float32),
        ],
    )(xp, posT, Wx, Wp, b2)


def _make_sc_edge(D, NP, E, CH, NC, NS):
    H = D // 2
    PPT = H // NS
    ES = E // NC
    U = 2
    mesh = plsc.VectorSubcoreMesh(
        core_axis_name="c", subcore_axis_name="s", num_cores=NC, num_subcores=NS
    )

    @functools.partial(
        pl.kernel,
        out_type=jax.ShapeDtypeStruct((NC * H * NP,), jnp.int32),
        mesh=mesh,
        compiler_params=pltpu.CompilerParams(needs_layout_passes=False),
        scratch_types=[
            pltpu.VMEM((PPT * NP,), jnp.int32),
            pltpu.VMEM((PPT * NP,), jnp.int32),
            pltpu.VMEM((CH,), jnp.int32),
            pltpu.VMEM((CH,), jnp.int32),
            pltpu.VMEM((CH,), jnp.int32),
            pltpu.VMEM((CH,), jnp.int32),
            pltpu.SemaphoreType.DMA,
            pltpu.SemaphoreType.DMA,
            pltpu.SemaphoreType.DMA,
            pltpu.SemaphoreType.DMA,
        ],
    )
    def agg(
        uP_hbm,
        src_hbm,
        dst_hbm,
        part_hbm,
        u_v,
        acc_v,
        src_a,
        src_b,
        dst_a,
        dst_b,
        sem_sa,
        sem_sb,
        sem_da,
        sem_db,
    ):
        shard = lax.axis_index("c")
        grp = lax.axis_index("s")
        fbase = grp * (PPT * NP)
        pltpu.sync_copy(uP_hbm.at[pl.ds(fbase, PPT * NP)], u_v)

        def tf_body(i, _):
            sl = pl.ds(i * _LANES, _LANES)
            k = _tf16(u_v[sl])
            u_v[sl] = k
            acc_v[sl] = k
            return 0

        lax.fori_loop(0, PPT * NP // _LANES, tf_body, 0)
        ebase = shard * ES

        def process(src_v, dst_v):
            def vec_body(i, _):
                idx_d, gw, aw = [], [], []
                for k in range(U):
                    base = (i * U + k) * _LANES
                    s16 = src_v[pl.ds(base, _LANES)]
                    d16 = dst_v[pl.ds(base, _LANES)]
                    for p in range(PPT):
                        idx_d.append(d16 + jnp.int32(p * NP))
                        gw.append(plsc.load_gather(u_v, [s16 + jnp.int32(p * NP)]))
                for j in range(U * PPT):
                    aw.append(plsc.load_gather(acc_v, [idx_d[j]]))
                mw = []
                for j in range(U * PPT):
                    mw.append(_maxw(gw[j], aw[j]))
                for j in range(U * PPT):
                    plsc.store_scatter(acc_v, [idx_d[j]], mw[j])
                bad = None
                for j in range(U * PPT):
                    rbw = plsc.load_gather(acc_v, [idx_d[j]])
                    c = rbw != mw[j]
                    bad = c if bad is None else (bad | c)

                def fix_cond(dirty):
                    return dirty

                def fix_body(_):
                    dirty = jnp.bool_(False)
                    for j in range(U * PPT):
                        rbw = plsc.load_gather(acc_v, [idx_d[j]])
                        w = _maxw(mw[j], rbw)
                        need = w != rbw
                        plsc.store_scatter(acc_v, [idx_d[j]], w, mask=need)
                        dirty = dirty | _anyv(need)
                    return dirty

                lax.while_loop(fix_cond, fix_body, _anyv(bad))
                return 0

            lax.fori_loop(0, CH // (U * _LANES), vec_body, 0)

        nchunks = ES // CH
        bufs = [(src_a, dst_a, sem_sa, sem_da), (src_b, dst_b, sem_sb, sem_db)]
        for bi, (sb, db, ss, sd) in enumerate(bufs):
            pltpu.async_copy(src_hbm.at[pl.ds(ebase + bi * CH, CH)], sb, ss)
            pltpu.async_copy(dst_hbm.at[pl.ds(ebase + bi * CH, CH)], db, sd)

        def outer_body(oi, _):
            for bi, (sb, db, ss, sd) in enumerate(bufs):
                ci = oi * 2 + bi
                pltpu.make_async_copy(
                    src_hbm.at[pl.ds(ebase + ci * CH, CH)], sb, ss
                ).wait()
                pltpu.make_async_copy(
                    dst_hbm.at[pl.ds(ebase + ci * CH, CH)], db, sd
                ).wait()
                process(sb, db)
                nci = lax.rem(ci + 2, nchunks)
                pltpu.async_copy(src_hbm.at[pl.ds(ebase + nci * CH, CH)], sb, ss)
                pltpu.async_copy(dst_hbm.at[pl.ds(ebase + nci * CH, CH)], db, sd)
            return 0

        lax.fori_loop(0, nchunks // 2, outer_body, 0)
        for bi, (sb, db, ss, sd) in enumerate(bufs):
            pltpu.make_async_copy(src_hbm.at[pl.ds(ebase + bi * CH, CH)], sb, ss).wait()
            pltpu.make_async_copy(dst_hbm.at[pl.ds(ebase + bi * CH, CH)], db, sd).wait()

        pltpu.sync_copy(acc_v, part_hbm.at[pl.ds(shard * (H * NP) + fbase, PPT * NP)])

    return agg


def _make_sc_merge(D, NP, NC, NS):
    H = D // 2
    NW = NC * NS
    PR = H // NW
    mesh = plsc.VectorSubcoreMesh(
        core_axis_name="c", subcore_axis_name="s", num_cores=NC, num_subcores=NS
    )

    @functools.partial(
        pl.kernel,
        out_type=jax.ShapeDtypeStruct((D * NP,), jnp.float32),
        mesh=mesh,
        compiler_params=pltpu.CompilerParams(needs_layout_passes=False),
        scratch_types=[
            pltpu.VMEM((PR * NP,), jnp.int32),
            pltpu.VMEM((PR * NP,), jnp.int32),
            pltpu.VMEM((2 * PR * NP,), jnp.float32),
        ],
    )
    def merge(part_hbm, vT_hbm, out_hbm, a0_v, a1_v, v_v):
        cid = lax.axis_index("c")
        sid = lax.axis_index("s")
        wid = sid * NC + cid
        pbase = wid * (PR * NP)
        pltpu.sync_copy(part_hbm.at[pl.ds(pbase, PR * NP)], a0_v)
        pltpu.sync_copy(part_hbm.at[pl.ds(H * NP + pbase, PR * NP)], a1_v)
        pltpu.sync_copy(vT_hbm.at[pl.ds(pbase, PR * NP)], v_v.at[pl.ds(0, PR * NP)])
        pltpu.sync_copy(
            vT_hbm.at[pl.ds(H * NP + pbase, PR * NP)],
            v_v.at[pl.ds(PR * NP, PR * NP)],
        )

        def body(i, _):
            sl = pl.ds(i * _LANES, _LANES)
            slh = pl.ds(PR * NP + i * _LANES, _LANES)
            w = _tf16(_maxw(a0_v[sl], a1_v[sl]))
            lo, hi = _unpk(w)
            v_v[sl] = lo - v_v[sl]
            v_v[slh] = hi - v_v[slh]
            return 0

        lax.fori_loop(0, PR * NP // _LANES, body, 0)
        pltpu.sync_copy(v_v.at[pl.ds(0, PR * NP)], out_hbm.at[pl.ds(pbase, PR * NP)])
        pltpu.sync_copy(
            v_v.at[pl.ds(PR * NP, PR * NP)],
            out_hbm.at[pl.ds(H * NP + pbase, PR * NP)],
        )

    return merge


def kernel(x, pos, edge_index, batch, W, b):
    N, D = x.shape
    E = edge_index.shape[1]
    NC, NS = 2, 16
    NW = NC * NS
    assert (D // 2) % NW == 0
    NP = -(-N // 256) * 256

    ES = E // NC
    CH = 0
    for cand in range(8192, 31, -32):
        if ES % cand == 0 and (ES // cand) % 2 == 0:
            CH = cand
            break
    assert CH > 0

    xp = jnp.pad(x, ((0, NP - N), (0, 0)))
    posT = jnp.pad(pos.T, ((0, 0), (0, NP - N)))
    Wx = W[:, :D]
    Wp = W[:, D:]
    b2 = b[:, None]

    uP, vT, p3T = _node_transform(xp, posT, Wx, Wp, b2, NP, D, 512)

    src = edge_index[0]
    dst = edge_index[1]
    agg = _make_sc_edge(D, NP, E, CH, NC, NS)
    part = agg(uP.reshape((D // 2) * NP), src, dst)
    merge = _make_sc_merge(D, NP, NC, NS)
    outF = merge(part, vT.reshape(D * NP))

    out = outF.reshape(D, NP)[:, :N].T
    pos3 = p3T[:, :N].T
    return (out, pos3, batch)
